# baseline (speedup 1.0000x reference)
import jax
import jax.numpy as jnp
from jax.experimental import pallas as pl
from jax.experimental.pallas import tpu as pltpu

_BN = 400
_DSPLIT = 2


def _body(src_ref, neigh_ref, wa_ref, ws_ref, out_ref, acc_ref):
    j = pl.program_id(1)
    partial = jnp.sum(neigh_ref[...], axis=1)

    @pl.when(j == 0)
    def _():
        acc_ref[...] = partial

    @pl.when(j > 0)
    def _():
        acc_ref[...] += partial

    @pl.when(j == _DSPLIT - 1)
    def _():
        deg = neigh_ref.shape[1] * _DSPLIT
        mean = acc_ref[...] * (1.0 / deg)
        h = jnp.dot(mean, wa_ref[...], preferred_element_type=jnp.float32)
        h += jnp.dot(src_ref[...], ws_ref[...], preferred_element_type=jnp.float32)
        out_ref[...] = jnp.maximum(h, 0.0)


def kernel(src_node_features, neighbor_node_features, W_agg, W_self):
    n, deg, d_in = neighbor_node_features.shape
    d_hid = W_agg.shape[1]
    bd = deg // _DSPLIT
    grid = (n // _BN, _DSPLIT)
    return pl.pallas_call(
        _body,
        grid=grid,
        in_specs=[
            pl.BlockSpec((_BN, d_in), lambda i, j: (i, 0)),
            pl.BlockSpec((_BN, bd, d_in), lambda i, j: (i, j, 0)),
            pl.BlockSpec((d_in, d_hid), lambda i, j: (0, 0)),
            pl.BlockSpec((d_in, d_hid), lambda i, j: (0, 0)),
        ],
        out_specs=pl.BlockSpec((_BN, d_hid), lambda i, j: (i, 0)),
        out_shape=jax.ShapeDtypeStruct((n, d_hid), jnp.float32),
        scratch_shapes=[pltpu.VMEM((_BN, d_in), jnp.float32)],
    )(src_node_features, neighbor_node_features, W_agg, W_self)


# speedup vs baseline: 1.3577x; 1.3577x over previous
import jax
import jax.numpy as jnp
from jax.experimental import pallas as pl

_BN = 400


def _body(src_ref, neigh_ref, wa_ref, ws_ref, out_ref):
    mean = jnp.mean(neigh_ref[...], axis=1)
    h = jnp.dot(mean, wa_ref[...], preferred_element_type=jnp.float32)
    h += jnp.dot(src_ref[...], ws_ref[...], preferred_element_type=jnp.float32)
    out_ref[...] = jnp.maximum(h, 0.0)


def kernel(src_node_features, neighbor_node_features, W_agg, W_self):
    n, deg, d_in = neighbor_node_features.shape
    d_hid = W_agg.shape[1]
    grid = (n // _BN,)
    return pl.pallas_call(
        _body,
        grid=grid,
        in_specs=[
            pl.BlockSpec((_BN, d_in), lambda i: (i, 0)),
            pl.BlockSpec((_BN, deg, d_in), lambda i: (i, 0, 0)),
            pl.BlockSpec((d_in, d_hid), lambda i: (0, 0)),
            pl.BlockSpec((d_in, d_hid), lambda i: (0, 0)),
        ],
        out_specs=pl.BlockSpec((_BN, d_hid), lambda i: (i, 0)),
        out_shape=jax.ShapeDtypeStruct((n, d_hid), jnp.float32),
    )(src_node_features, neighbor_node_features, W_agg, W_self)
